# baseline (device time: 38115 ns/iter reference)
import jax
import jax.numpy as jnp
from jax import lax
from jax.experimental import pallas as pl
from jax.experimental.pallas import tpu as pltpu


def kernel(Q, K, V):
    b, sq, h, d = Q.shape
    scale = d ** -0.5

    def body(q_ref, k_ref, v_ref, out_ref, o_buf, ml_buf, send_sems, recv_sems):
        my_x = lax.axis_index("x")
        my_y = lax.axis_index("y")
        peer = (my_x, 1 - my_y)

        barrier = pltpu.get_barrier_semaphore()
        pl.semaphore_signal(
            barrier, inc=1, device_id=peer, device_id_type=pl.DeviceIdType.MESH
        )
        pl.semaphore_wait(barrier, 1)

        ms, ls, os_ = [], [], []
        for bi in range(b):
            qb = q_ref[bi, 0].astype(jnp.float32)
            kb = k_ref[bi].astype(jnp.float32)
            vb = v_ref[bi].astype(jnp.float32)
            s = jnp.sum(kb * qb[None], axis=-1) * scale
            m = jnp.max(s, axis=0, keepdims=True)
            p = jnp.exp(s - m)
            l = jnp.sum(p, axis=0, keepdims=True)
            o = jnp.sum(p[:, :, None] * vb, axis=0)
            ms.append(m)
            ls.append(l)
            os_.append(o[None])
        m_loc = jnp.concatenate(ms, axis=0)
        l_loc = jnp.concatenate(ls, axis=0)
        o_loc = jnp.concatenate(os_, axis=0)

        o_buf[0] = o_loc
        ml_buf[0, 0] = m_loc
        ml_buf[0, 1] = l_loc

        rdma_o = pltpu.make_async_remote_copy(
            src_ref=o_buf.at[0],
            dst_ref=o_buf.at[1],
            send_sem=send_sems.at[0],
            recv_sem=recv_sems.at[0],
            device_id=peer,
            device_id_type=pl.DeviceIdType.MESH,
        )
        rdma_ml = pltpu.make_async_remote_copy(
            src_ref=ml_buf.at[0],
            dst_ref=ml_buf.at[1],
            send_sem=send_sems.at[1],
            recv_sem=recv_sems.at[1],
            device_id=peer,
            device_id_type=pl.DeviceIdType.MESH,
        )
        rdma_o.start()
        rdma_ml.start()
        rdma_o.wait()
        rdma_ml.wait()

        m_rem = ml_buf[1, 0]
        l_rem = ml_buf[1, 1]
        o_rem = o_buf[1]
        m_tot = jnp.maximum(m_loc, m_rem)
        a_loc = jnp.exp(m_loc - m_tot)
        a_rem = jnp.exp(m_rem - m_tot)
        l_tot = l_loc * a_loc + l_rem * a_rem
        o_tot = o_loc * a_loc[:, :, None] + o_rem * a_rem[:, :, None]
        out_ref[:, 0] = o_tot / l_tot[:, :, None]

    return pl.pallas_call(
        body,
        out_shape=jax.ShapeDtypeStruct((b, sq, h, d), jnp.float32),
        in_specs=[pl.BlockSpec(memory_space=pltpu.VMEM)] * 3,
        out_specs=pl.BlockSpec(memory_space=pltpu.VMEM),
        scratch_shapes=[
            pltpu.VMEM((2, b, h, d), jnp.float32),
            pltpu.VMEM((2, 2, b, h), jnp.float32),
            pltpu.SemaphoreType.DMA((2,)),
            pltpu.SemaphoreType.DMA((2,)),
        ],
        compiler_params=pltpu.CompilerParams(collective_id=0),
    )(Q, K, V)
